# Initial kernel scaffold; baseline (speedup 1.0000x reference)
import jax
import jax.numpy as jnp
from jax import lax
from jax.experimental import pallas as pl
from jax.experimental.pallas import tpu as pltpu
from jax.experimental.pallas import tpu_sc as plsc

N = 10000
NPAD = 10240
E = 320000
D = 128
G = 64
C = 2
CH = 80
CPT = 125
RPT = 625
EPS = 1e-5

_MESH = plsc.VectorSubcoreMesh(core_axis_name="c", subcore_axis_name="s")
_F32 = jnp.float32


def _deg_body(dstr, ewr, degp, didx_v, ewv, ibuf, dacc):
    cid = lax.axis_index("c")
    sid = lax.axis_index("s")
    wid = cid * 16 + sid
    val = jnp.where(cid == 0, 1.0, 0.0).astype(_F32)
    ones = jnp.full((16,), 1.0, _F32) * val
    for i in range(40):
        ibuf[pl.ds(i * 16, 16)] = ones
    pltpu.sync_copy(ibuf, dacc.at[pl.ds(sid * 640, 640)])
    pltpu.sync_copy(dstr.at[pl.ds(wid * CPT, CPT)], didx_v)
    pltpu.sync_copy(ewr.at[pl.ds(wid * CPT, CPT)], ewv)
    plsc.subcore_barrier()

    def chunk(g, carry):
        pltpu.sync_copy(ewv.at[g], dacc.at[didx_v.at[g]], add=True)
        return carry

    lax.fori_loop(0, CPT, chunk, 0)
    plsc.subcore_barrier()
    pltpu.sync_copy(dacc.at[pl.ds(sid * 640, 640)],
                    degp.at[cid, pl.ds(sid * 640, 640)])


_deg_call = pl.kernel(
    _deg_body,
    out_type=jax.ShapeDtypeStruct((2, NPAD), _F32),
    mesh=_MESH,
    scratch_types=[
        pltpu.VMEM((CPT, CH), jnp.int32),
        pltpu.VMEM((CPT, CH), _F32),
        pltpu.VMEM((640,), _F32),
        pltpu.VMEM_SHARED((NPAD,), _F32),
    ],
)


def _scat_body(h, srcr, dstr, ewr, dinv, out,
               dinv_v, sidx_v, didx_v, ewv, normv, rows, zbuf, acc, sem):
    cid = lax.axis_index("c")
    sid = lax.axis_index("s")
    wid = cid * 16 + sid
    pltpu.sync_copy(dinv, dinv_v)
    z16 = jnp.zeros((16,), _F32)

    def zrow(i, carry):
        for q in range(8):
            zbuf[i, pl.ds(q * 16, 16)] = z16
        return carry

    lax.fori_loop(0, 125, zrow, 0)
    for k in range(5):
        pltpu.sync_copy(zbuf, acc.at[pl.ds(sid * RPT + k * 125, 125)])
    pltpu.sync_copy(srcr.at[pl.ds(wid * CPT, CPT)], sidx_v)
    pltpu.sync_copy(dstr.at[pl.ds(wid * CPT, CPT)], didx_v)
    pltpu.sync_copy(ewr.at[pl.ds(wid * CPT, CPT)], ewv)
    plsc.subcore_barrier()

    def chunk(g, carry):
        pltpu.async_copy(h.at[sidx_v.at[g]], rows, sem).wait()
        for j in range(5):
            iv = sidx_v[g, pl.ds(j * 16, 16)]
            dv = didx_v[g, pl.ds(j * 16, 16)]
            a = plsc.load_gather(dinv_v, [iv])
            b = plsc.load_gather(dinv_v, [dv])
            normv[pl.ds(j * 16, 16)] = a * ewv[g, pl.ds(j * 16, 16)] * b

        def scale(e, c2):
            ne = normv[e]
            for q in range(8):
                rows[e, pl.ds(q * 16, 16)] = rows[e, pl.ds(q * 16, 16)] * ne
            return c2

        lax.fori_loop(0, CH, scale, 0)
        pltpu.sync_copy(rows, acc.at[didx_v.at[g]], add=True)
        return carry

    lax.fori_loop(0, CPT, chunk, 0)
    plsc.subcore_barrier()
    for k in range(5):
        r0 = sid * RPT + k * 125
        pltpu.sync_copy(acc.at[pl.ds(r0, 125)], out.at[cid, pl.ds(r0, 125)])


_scat_call = pl.kernel(
    _scat_body,
    out_type=jax.ShapeDtypeStruct((2, N, D), _F32),
    mesh=_MESH,
    scratch_types=[
        pltpu.VMEM((NPAD,), _F32),
        pltpu.VMEM((CPT, CH), jnp.int32),
        pltpu.VMEM((CPT, CH), jnp.int32),
        pltpu.VMEM((CPT, CH), _F32),
        pltpu.VMEM((CH,), _F32),
        pltpu.VMEM((CH, D), _F32),
        pltpu.VMEM((125, D), _F32),
        pltpu.VMEM_SHARED((N, D), _F32),
        pltpu.SemaphoreType.DMA,
    ],
)


def _h0_body(x_ref, w_ref, degp_ref, h0_ref, dinv_ref):
    d = degp_ref[0:80, :] + degp_ref[80:160, :]
    dinv_ref[...] = jnp.where(d > 0, lax.rsqrt(d), 0.0)
    h0_ref[...] = jnp.dot(x_ref[...], w_ref[...],
                          preferred_element_type=_F32)


_h0_call = pl.pallas_call(
    _h0_body,
    out_shape=[jax.ShapeDtypeStruct((N, D), _F32),
               jax.ShapeDtypeStruct((80, 128), _F32)],
)


def _epi1_body(part_ref, h_ref, dinv_ref, b_ref, g_ref, bt_ref, w_ref,
               h1_ref):
    dv = dinv_ref[...]
    y = part_ref[0] + part_ref[1] + h_ref[...] * (dv * dv) + b_ref[...]
    y = jnp.maximum(y, 0.0)
    m = jnp.mean(y, axis=0, keepdims=True)
    v = jnp.mean(y * y, axis=0, keepdims=True) - m * m
    a = g_ref[...] * lax.rsqrt(v + EPS)
    z = (y - m) * a + bt_ref[...]
    h1_ref[...] = jnp.dot(z, w_ref[...], preferred_element_type=_F32)


_epi1_call = pl.pallas_call(
    _epi1_body,
    out_shape=jax.ShapeDtypeStruct((N, D), _F32),
)


def _epi2_body(part_ref, h_ref, dinv_ref, b_ref, g_ref, bt_ref, batch_ref,
               wl1_ref, bl1_ref, wl2_ref, bl2_ref, out_ref):
    dv = dinv_ref[...]
    y = part_ref[0] + part_ref[1] + h_ref[...] * (dv * dv) + b_ref[...]
    y = jnp.maximum(y, 0.0)
    m = jnp.mean(y, axis=0, keepdims=True)
    v = jnp.mean(y * y, axis=0, keepdims=True) - m * m
    a = g_ref[...] * lax.rsqrt(v + EPS)
    z = (y - m) * a + bt_ref[...]
    oh = (batch_ref[...] == lax.broadcasted_iota(jnp.int32, (G, 1), 0))
    pooled = jnp.dot(oh.astype(_F32), z, preferred_element_type=_F32)
    h2 = jnp.maximum(
        jnp.dot(pooled, wl1_ref[...], preferred_element_type=_F32)
        + bl1_ref[...], 0.0)
    logits = (jnp.dot(h2, wl2_ref[...], preferred_element_type=_F32)
              + bl2_ref[...])
    mx = jnp.max(logits, axis=-1, keepdims=True)
    sh = logits - mx
    out_ref[...] = sh - jnp.log(jnp.sum(jnp.exp(sh), axis=-1, keepdims=True))


_epi2_call = pl.pallas_call(
    _epi2_body,
    out_shape=jax.ShapeDtypeStruct((G, C), _F32),
)


def kernel(x, edge_index, batch, edge_attr,
           W0, b0, W1, b1, g0, bt0, g1, bt1, Wl1, bl1, Wl2, bl2):
    src = edge_index[0].astype(jnp.int32)
    dst = edge_index[1].astype(jnp.int32)
    ew = edge_attr[:, 0]
    srcr = src.reshape(E // CH, CH)
    dstr = dst.reshape(E // CH, CH)
    ewr = ew.reshape(E // CH, CH)

    degp = _deg_call(dstr, ewr)
    h0, dinvm = _h0_call(x, W0, degp.reshape(160, 128))
    dinv_flat = dinvm.reshape(NPAD)
    dinv_col = dinv_flat[:N].reshape(N, 1)

    part1 = _scat_call(h0, srcr, dstr, ewr, dinv_flat)
    h1 = _epi1_call(part1, h0, dinv_col, b0.reshape(1, D),
                    g0.reshape(1, D), bt0.reshape(1, D), W1)
    part2 = _scat_call(h1, srcr, dstr, ewr, dinv_flat)
    out = _epi2_call(part2, h1, dinv_col, b1.reshape(1, D),
                     g1.reshape(1, D), bt1.reshape(1, D),
                     batch.astype(jnp.int32).reshape(1, N),
                     Wl1, bl1.reshape(1, D), Wl2, bl2.reshape(1, C))
    return out


# speedup vs baseline: 12.9540x; 12.9540x over previous
import jax
import jax.numpy as jnp
from jax import lax
from jax.experimental import pallas as pl
from jax.experimental.pallas import tpu as pltpu
from jax.experimental.pallas import tpu_sc as plsc

N = 10000
NROWS = 10240
NPAD = 16384
E = 320000
D = 128
G = 64
C = 2
CH = 80
CPT = 128
BC = 16
NB = CPT // BC
EPADT = 32 * CPT * CH
EPS = 1e-5

_MESH = plsc.VectorSubcoreMesh(core_axis_name="c", subcore_axis_name="s")
_F32 = jnp.float32
_SC_PARAMS = pltpu.CompilerParams(needs_layout_passes=False)


def _deg_body(dstr, ewr, deg0, deg1, didx_v, ewv, ibuf, dacc):
    cid = lax.axis_index("c")
    sid = lax.axis_index("s")
    wid = cid * 16 + sid
    val = jnp.where(cid == 0, 1.0, 0.0).astype(_F32)
    ones = jnp.full((16,), 1.0, _F32) * val
    for i in range(64):
        ibuf[pl.ds(i * 16, 16)] = ones
    pltpu.sync_copy(ibuf, dacc.at[pl.ds(sid * 1024, 1024)])
    plsc.subcore_barrier()

    def block(b, carry):
        pltpu.sync_copy(dstr.at[wid, pl.ds(b * BC, BC)], didx_v)
        pltpu.sync_copy(ewr.at[wid, pl.ds(b * BC, BC)], ewv)

        def chunk(g, c1):
            pltpu.sync_copy(ewv.at[g], dacc.at[didx_v.at[g]], add=True)
            return c1

        lax.fori_loop(0, BC, chunk, 0)
        return carry

    lax.fori_loop(0, NB, block, 0)
    plsc.subcore_barrier()

    @pl.when(cid == 0)
    def _():
        pltpu.sync_copy(dacc.at[pl.ds(sid * 1024, 1024)],
                        deg0.at[pl.ds(sid * 1024, 1024)])

    @pl.when(cid == 1)
    def _():
        pltpu.sync_copy(dacc.at[pl.ds(sid * 1024, 1024)],
                        deg1.at[pl.ds(sid * 1024, 1024)])


_deg_call = pl.kernel(
    _deg_body,
    out_type=[jax.ShapeDtypeStruct((NPAD,), _F32),
              jax.ShapeDtypeStruct((NPAD,), _F32)],
    mesh=_MESH,
    compiler_params=_SC_PARAMS,
    scratch_types=[
        pltpu.VMEM((BC, CH), jnp.int32),
        pltpu.VMEM((BC, CH), _F32),
        pltpu.VMEM((1024,), _F32),
        pltpu.VMEM_SHARED((NPAD,), _F32),
    ],
)


def _scat_body(h, srcr, dstr, ewr, dinv, part0, part1,
               dinv_v, sidx_v, didx_v, ewv, normv, rows, acc, sem):
    cid = lax.axis_index("c")
    sid = lax.axis_index("s")
    wid = cid * 16 + sid
    pltpu.sync_copy(dinv.at[pl.ds(0, NROWS)], dinv_v)
    z16 = jnp.zeros((16,), _F32)

    def zrow(i, carry):
        for q in range(8):
            rows[i, pl.ds(q * 16, 16)] = z16
        return carry

    lax.fori_loop(0, CH, zrow, 0)
    for k in range(8):
        pltpu.sync_copy(rows, acc.at[pl.ds(sid * 640 + k * CH, CH)])
    plsc.subcore_barrier()

    def block(b, carry):
        pltpu.sync_copy(srcr.at[wid, pl.ds(b * BC, BC)], sidx_v)
        pltpu.sync_copy(dstr.at[wid, pl.ds(b * BC, BC)], didx_v)
        pltpu.sync_copy(ewr.at[wid, pl.ds(b * BC, BC)], ewv)

        def chunk(g, c1):
            pltpu.async_copy(h.at[sidx_v.at[g]], rows, sem).wait()
            for j in range(5):
                iv = sidx_v[g, pl.ds(j * 16, 16)]
                dv = didx_v[g, pl.ds(j * 16, 16)]
                a = plsc.load_gather(dinv_v, [iv])
                bb = plsc.load_gather(dinv_v, [dv])
                normv[pl.ds(j * 16, 16)] = a * ewv[g, pl.ds(j * 16, 16)] * bb

            def scale(e, c2):
                ne = plsc.load_gather(normv, [jnp.full((16,), e, jnp.int32)])
                for q in range(8):
                    rows[e, pl.ds(q * 16, 16)] = (
                        rows[e, pl.ds(q * 16, 16)] * ne)
                return c2

            lax.fori_loop(0, CH, scale, 0)
            pltpu.sync_copy(rows, acc.at[didx_v.at[g]], add=True)
            return c1

        lax.fori_loop(0, BC, chunk, 0)
        return carry

    lax.fori_loop(0, NB, block, 0)
    plsc.subcore_barrier()

    @pl.when(cid == 0)
    def _():
        pltpu.sync_copy(acc.at[pl.ds(sid * 640, 640)],
                        part0.at[pl.ds(sid * 640, 640)])

    @pl.when(cid == 1)
    def _():
        pltpu.sync_copy(acc.at[pl.ds(sid * 640, 640)],
                        part1.at[pl.ds(sid * 640, 640)])


_scat_call = pl.kernel(
    _scat_body,
    out_type=[jax.ShapeDtypeStruct((NROWS, D), _F32),
              jax.ShapeDtypeStruct((NROWS, D), _F32)],
    mesh=_MESH,
    compiler_params=_SC_PARAMS,
    scratch_types=[
        pltpu.VMEM((NROWS,), _F32),
        pltpu.VMEM((BC, CH), jnp.int32),
        pltpu.VMEM((BC, CH), jnp.int32),
        pltpu.VMEM((BC, CH), _F32),
        pltpu.VMEM((CH,), _F32),
        pltpu.VMEM((CH, D), _F32),
        pltpu.VMEM_SHARED((NROWS, D), _F32),
        pltpu.SemaphoreType.DMA,
    ],
)


def _h0_body(x_ref, w_ref, deg0_ref, deg1_ref, h0_ref, dinv_ref):
    d = deg0_ref[...] + deg1_ref[...]
    dinv_ref[...] = jnp.where(d > 0, lax.rsqrt(d), 0.0)
    h0_ref[...] = jnp.dot(x_ref[...], w_ref[...],
                          preferred_element_type=_F32)


_h0_call = pl.pallas_call(
    _h0_body,
    out_shape=[jax.ShapeDtypeStruct((N, D), _F32),
               jax.ShapeDtypeStruct((128, 128), _F32)],
)


def _epi1_body(p0_ref, p1_ref, h_ref, dinv_ref, b_ref, g_ref, bt_ref,
               w_ref, h1_ref):
    dv = dinv_ref[...]
    y = (p0_ref[0:N, :] + p1_ref[0:N, :]
         + h_ref[...] * (dv * dv) + b_ref[...])
    y = jnp.maximum(y, 0.0)
    m = jnp.mean(y, axis=0, keepdims=True)
    v = jnp.mean(y * y, axis=0, keepdims=True) - m * m
    a = g_ref[...] * lax.rsqrt(v + EPS)
    z = (y - m) * a + bt_ref[...]
    h1_ref[...] = jnp.dot(z, w_ref[...], preferred_element_type=_F32)


_epi1_call = pl.pallas_call(
    _epi1_body,
    out_shape=jax.ShapeDtypeStruct((N, D), _F32),
)


def _epi2_body(p0_ref, p1_ref, h_ref, dinv_ref, b_ref, g_ref, bt_ref,
               batch_ref, wl1_ref, bl1_ref, wl2_ref, bl2_ref, out_ref):
    dv = dinv_ref[...]
    y = (p0_ref[0:N, :] + p1_ref[0:N, :]
         + h_ref[...] * (dv * dv) + b_ref[...])
    y = jnp.maximum(y, 0.0)
    m = jnp.mean(y, axis=0, keepdims=True)
    v = jnp.mean(y * y, axis=0, keepdims=True) - m * m
    a = g_ref[...] * lax.rsqrt(v + EPS)
    z = (y - m) * a + bt_ref[...]
    oh = (batch_ref[...] == lax.broadcasted_iota(jnp.int32, (G, 1), 0))
    pooled = jnp.dot(oh.astype(_F32), z, preferred_element_type=_F32,
                     precision=lax.Precision.HIGHEST)
    h2 = jnp.maximum(
        jnp.dot(pooled, wl1_ref[...], preferred_element_type=_F32)
        + bl1_ref[...], 0.0)
    logits = (jnp.dot(h2, wl2_ref[...], preferred_element_type=_F32)
              + bl2_ref[...])
    mx = jnp.max(logits, axis=-1, keepdims=True)
    sh = logits - mx
    out_ref[...] = sh - jnp.log(jnp.sum(jnp.exp(sh), axis=-1, keepdims=True))


_epi2_call = pl.pallas_call(
    _epi2_body,
    out_shape=jax.ShapeDtypeStruct((G, C), _F32),
)


def kernel(x, edge_index, batch, edge_attr,
           W0, b0, W1, b1, g0, bt0, g1, bt1, Wl1, bl1, Wl2, bl2):
    src = edge_index[0].astype(jnp.int32)
    dst = edge_index[1].astype(jnp.int32)
    ew = edge_attr[:, 0]
    pad = EPADT - E
    pad_idx = (jnp.arange(pad, dtype=jnp.int32) * 37) % N
    src_p = jnp.concatenate([src, pad_idx])
    dst_p = jnp.concatenate([dst, pad_idx])
    ew_p = jnp.concatenate([ew, jnp.zeros((pad,), _F32)])
    srcr = src_p.reshape(32, CPT, CH)
    dstr = dst_p.reshape(32, CPT, CH)
    ewr = ew_p.reshape(32, CPT, CH)

    deg0, deg1 = _deg_call(dstr, ewr)
    h0, dinvm = _h0_call(x, W0, deg0.reshape(128, 128),
                         deg1.reshape(128, 128))
    dinv_flat = dinvm.reshape(NPAD)
    dinv_col = dinv_flat[:N].reshape(N, 1)

    p0, p1 = _scat_call(h0, srcr, dstr, ewr, dinv_flat)
    h1 = _epi1_call(p0, p1, h0, dinv_col, b0.reshape(1, D),
                    g0.reshape(1, D), bt0.reshape(1, D), W1)
    q0, q1 = _scat_call(h1, srcr, dstr, ewr, dinv_flat)
    out = _epi2_call(q0, q1, h1, dinv_col, b1.reshape(1, D),
                     g1.reshape(1, D), bt1.reshape(1, D),
                     batch.astype(jnp.int32).reshape(1, N),
                     Wl1, bl1.reshape(1, D), Wl2, bl2.reshape(1, C))
    return out


# speedup vs baseline: 20.8587x; 1.6102x over previous
import jax
import jax.numpy as jnp
from jax import lax
from jax.experimental import pallas as pl
from jax.experimental.pallas import tpu as pltpu
from jax.experimental.pallas import tpu_sc as plsc

N = 10000
NROWS = 10240
NPAD = 16384
E = 320000
D = 128
G = 64
C = 2
CH = 64
CPT = 160
EPADT = 32 * CPT * CH
EPS = 1e-5

_MESH = plsc.VectorSubcoreMesh(core_axis_name="c", subcore_axis_name="s")
_F32 = jnp.float32
_SC_PARAMS = pltpu.CompilerParams(needs_layout_passes=False)


def _deg_body(dstr, ewr, deg0, deg1, didx_v, ewv, ibuf, dacc):
    cid = lax.axis_index("c")
    sid = lax.axis_index("s")
    wid = cid * 16 + sid
    val = jnp.where(cid == 0, 1.0, 0.0).astype(_F32)
    ones = jnp.full((16,), 1.0, _F32) * val
    for i in range(64):
        ibuf[pl.ds(i * 16, 16)] = ones
    pltpu.sync_copy(ibuf, dacc.at[pl.ds(sid * 1024, 1024)])
    pltpu.sync_copy(dstr.at[wid], didx_v)
    pltpu.sync_copy(ewr.at[wid], ewv)
    plsc.subcore_barrier()

    def chunk(g, c1):
        pltpu.sync_copy(ewv.at[g], dacc.at[didx_v.at[g]], add=True)
        return c1

    lax.fori_loop(0, CPT, chunk, 0)
    plsc.subcore_barrier()

    @pl.when(cid == 0)
    def _():
        pltpu.sync_copy(dacc.at[pl.ds(sid * 1024, 1024)],
                        deg0.at[pl.ds(sid * 1024, 1024)])

    @pl.when(cid == 1)
    def _():
        pltpu.sync_copy(dacc.at[pl.ds(sid * 1024, 1024)],
                        deg1.at[pl.ds(sid * 1024, 1024)])


_deg_call = pl.kernel(
    _deg_body,
    out_type=[jax.ShapeDtypeStruct((NPAD,), _F32),
              jax.ShapeDtypeStruct((NPAD,), _F32)],
    mesh=_MESH,
    compiler_params=_SC_PARAMS,
    scratch_types=[
        pltpu.VMEM((CPT, CH), jnp.int32),
        pltpu.VMEM((CPT, CH), _F32),
        pltpu.VMEM((1024,), _F32),
        pltpu.VMEM_SHARED((NPAD,), _F32),
    ],
)


def _scale_scatter(rows, ewst, acc, didx_v, g):
    for j in range(CH // 16):
        nv = ewst[g, pl.ds(j * 16, 16)]
        for t in range(16):
            e = j * 16 + t
            ne = nv[t]
            for q in range(8):
                rows[e, pl.ds(q * 16, 16)] = rows[e, pl.ds(q * 16, 16)] * ne
    pltpu.sync_copy(rows, acc.at[didx_v.at[g]], add=True)


def _scat_body(hs, srcr, dstr, ewr, part0, part1,
               sidx_v, didx_v, ewst, rows_a, rows_b, acc, sem_a, sem_b):
    cid = lax.axis_index("c")
    sid = lax.axis_index("s")
    wid = cid * 16 + sid
    z16 = jnp.zeros((16,), _F32)

    def zrow(i, carry):
        for q in range(8):
            rows_a[i, pl.ds(q * 16, 16)] = z16
        return carry

    lax.fori_loop(0, CH, zrow, 0)
    for k in range(10):
        pltpu.sync_copy(rows_a, acc.at[pl.ds(sid * 640 + k * CH, CH)])
    plsc.subcore_barrier()

    HC = CPT // 2
    for half in range(2):
        pltpu.sync_copy(srcr.at[wid, pl.ds(half * HC, HC)], sidx_v)
        pltpu.sync_copy(dstr.at[wid, pl.ds(half * HC, HC)], didx_v)
        pltpu.sync_copy(ewr.at[wid, pl.ds(half * HC, HC)], ewst)
        pltpu.async_copy(hs.at[sidx_v.at[0]], rows_a, sem_a)

        def pair(p, carry):
            g0 = 2 * p
            g1 = g0 + 1
            pltpu.make_async_copy(hs.at[sidx_v.at[g0]], rows_a,
                                  sem_a).wait()
            pltpu.async_copy(hs.at[sidx_v.at[g1]], rows_b, sem_b)
            _scale_scatter(rows_a, ewst, acc, didx_v, g0)
            pltpu.make_async_copy(hs.at[sidx_v.at[g1]], rows_b,
                                  sem_b).wait()

            @pl.when(g0 + 2 < HC)
            def _():
                pltpu.async_copy(hs.at[sidx_v.at[g0 + 2]], rows_a, sem_a)

            _scale_scatter(rows_b, ewst, acc, didx_v, g1)
            return carry

        lax.fori_loop(0, HC // 2, pair, 0)
    plsc.subcore_barrier()

    @pl.when(cid == 0)
    def _():
        pltpu.sync_copy(acc.at[pl.ds(sid * 640, 640)],
                        part0.at[pl.ds(sid * 640, 640)])

    @pl.when(cid == 1)
    def _():
        pltpu.sync_copy(acc.at[pl.ds(sid * 640, 640)],
                        part1.at[pl.ds(sid * 640, 640)])


_scat_call = pl.kernel(
    _scat_body,
    out_type=[jax.ShapeDtypeStruct((NROWS, D), _F32),
              jax.ShapeDtypeStruct((NROWS, D), _F32)],
    mesh=_MESH,
    compiler_params=_SC_PARAMS,
    scratch_types=[
        pltpu.VMEM((CPT // 2, CH), jnp.int32),
        pltpu.VMEM((CPT // 2, CH), jnp.int32),
        pltpu.VMEM((CPT // 2, CH), _F32),
        pltpu.VMEM((CH, D), _F32),
        pltpu.VMEM((CH, D), _F32),
        pltpu.VMEM_SHARED((NROWS, D), _F32),
        pltpu.SemaphoreType.DMA,
        pltpu.SemaphoreType.DMA,
    ],
)


def _h0_body(x_ref, w_ref, deg0_ref, deg1_ref, hs0_ref, dinv_ref):
    d = deg0_ref[...] + deg1_ref[...]
    dv = jnp.where(d > 0, lax.rsqrt(d), 0.0)
    dinv_ref[...] = dv
    hs0_ref[...] = jnp.dot(x_ref[...], w_ref[...],
                           preferred_element_type=_F32) * dv


_h0_call = pl.pallas_call(
    _h0_body,
    out_shape=[jax.ShapeDtypeStruct((N, D), _F32),
               jax.ShapeDtypeStruct((N, 1), _F32)],
)


def _epi1_body(p0_ref, p1_ref, hs_ref, dinv_ref, b_ref, g_ref, bt_ref,
               w_ref, hs1_ref):
    dv = dinv_ref[...]
    y = (p0_ref[0:N, :] + p1_ref[0:N, :] + hs_ref[...]) * dv + b_ref[...]
    y = jnp.maximum(y, 0.0)
    m = jnp.mean(y, axis=0, keepdims=True)
    v = jnp.mean(y * y, axis=0, keepdims=True) - m * m
    a = g_ref[...] * lax.rsqrt(v + EPS)
    z = (y - m) * a + bt_ref[...]
    hs1_ref[...] = jnp.dot(z, w_ref[...], preferred_element_type=_F32) * dv


_epi1_call = pl.pallas_call(
    _epi1_body,
    out_shape=jax.ShapeDtypeStruct((N, D), _F32),
)


def _epi2_body(p0_ref, p1_ref, hs_ref, dinv_ref, b_ref, g_ref, bt_ref,
               batch_ref, wl1_ref, bl1_ref, wl2_ref, bl2_ref, out_ref):
    dv = dinv_ref[...]
    y = (p0_ref[0:N, :] + p1_ref[0:N, :] + hs_ref[...]) * dv + b_ref[...]
    y = jnp.maximum(y, 0.0)
    m = jnp.mean(y, axis=0, keepdims=True)
    v = jnp.mean(y * y, axis=0, keepdims=True) - m * m
    a = g_ref[...] * lax.rsqrt(v + EPS)
    z = (y - m) * a + bt_ref[...]
    oh = (batch_ref[...] == lax.broadcasted_iota(jnp.int32, (G, 1), 0))
    pooled = jnp.dot(oh.astype(_F32), z, preferred_element_type=_F32,
                     precision=lax.Precision.HIGHEST)
    h2 = jnp.maximum(
        jnp.dot(pooled, wl1_ref[...], preferred_element_type=_F32)
        + bl1_ref[...], 0.0)
    logits = (jnp.dot(h2, wl2_ref[...], preferred_element_type=_F32)
              + bl2_ref[...])
    mx = jnp.max(logits, axis=-1, keepdims=True)
    sh = logits - mx
    out_ref[...] = sh - jnp.log(jnp.sum(jnp.exp(sh), axis=-1, keepdims=True))


_epi2_call = pl.pallas_call(
    _epi2_body,
    out_shape=jax.ShapeDtypeStruct((G, C), _F32),
)


def kernel(x, edge_index, batch, edge_attr,
           W0, b0, W1, b1, g0, bt0, g1, bt1, Wl1, bl1, Wl2, bl2):
    src = edge_index[0].astype(jnp.int32)
    dst = edge_index[1].astype(jnp.int32)
    ew = edge_attr[:, 0]
    pad = EPADT - E
    pad_idx = (jnp.arange(pad, dtype=jnp.int32) * 37) % N
    src_p = jnp.concatenate([src, pad_idx])
    dst_p = jnp.concatenate([dst, pad_idx])
    ew_p = jnp.concatenate([ew, jnp.zeros((pad,), _F32)])
    srcr = src_p.reshape(32, CPT, CH)
    dstr = dst_p.reshape(32, CPT, CH)
    ewr = ew_p.reshape(32, CPT, CH)

    deg0, deg1 = _deg_call(dstr, ewr)
    hs0, dinv_col = _h0_call(x, W0, deg0[:N].reshape(N, 1),
                             deg1[:N].reshape(N, 1))

    p0, p1 = _scat_call(hs0, srcr, dstr, ewr)
    hs1 = _epi1_call(p0, p1, hs0, dinv_col, b0.reshape(1, D),
                     g0.reshape(1, D), bt0.reshape(1, D), W1)
    q0, q1 = _scat_call(hs1, srcr, dstr, ewr)
    out = _epi2_call(q0, q1, hs1, dinv_col, b1.reshape(1, D),
                     g1.reshape(1, D), bt1.reshape(1, D),
                     batch.astype(jnp.int32).reshape(1, N),
                     Wl1, bl1.reshape(1, D), Wl2, bl2.reshape(1, C))
    return out


# speedup vs baseline: 21.1458x; 1.0138x over previous
import jax
import jax.numpy as jnp
import numpy as np
from jax import lax
from jax.experimental import pallas as pl
from jax.experimental.pallas import tpu as pltpu
from jax.experimental.pallas import tpu_sc as plsc

N = 10000
NROWS = 10240
NPAD = 16384
E = 320000
D = 128
G = 64
C = 2
CH = 64
CPT = 160
EPADT = 32 * CPT * CH
EPS = 1e-5

_q = np.arange(D) // 32
_r = np.arange(D) % 32
PHI = (32 * _q + _r // 2 + 16 * (_r % 2)).astype(np.int32)

_MESH = plsc.VectorSubcoreMesh(core_axis_name="c", subcore_axis_name="s")
_F32 = jnp.float32
_BF16 = jnp.bfloat16
_SC_PARAMS = pltpu.CompilerParams(needs_layout_passes=False,
                                  use_tc_tiling_on_sc=False)


def _deg_body(dstr, ewr, deg0, deg1, didx_v, ewv, ibuf, dacc):
    cid = lax.axis_index("c")
    sid = lax.axis_index("s")
    wid = cid * 16 + sid
    val = jnp.where(cid == 0, 1.0, 0.0).astype(_F32)
    ones = jnp.full((16,), 1.0, _F32) * val
    for i in range(64):
        ibuf[pl.ds(i * 16, 16)] = ones
    pltpu.sync_copy(ibuf, dacc.at[pl.ds(sid * 1024, 1024)])
    pltpu.sync_copy(dstr.at[wid], didx_v)
    pltpu.sync_copy(ewr.at[wid], ewv)
    plsc.subcore_barrier()

    def chunk(g, c1):
        pltpu.sync_copy(ewv.at[g], dacc.at[didx_v.at[g]], add=True)
        return c1

    lax.fori_loop(0, CPT, chunk, 0)
    plsc.subcore_barrier()

    @pl.when(cid == 0)
    def _():
        pltpu.sync_copy(dacc.at[pl.ds(sid * 1024, 1024)],
                        deg0.at[pl.ds(sid * 1024, 1024)])

    @pl.when(cid == 1)
    def _():
        pltpu.sync_copy(dacc.at[pl.ds(sid * 1024, 1024)],
                        deg1.at[pl.ds(sid * 1024, 1024)])


_deg_call = pl.kernel(
    _deg_body,
    out_type=[jax.ShapeDtypeStruct((NPAD,), _F32),
              jax.ShapeDtypeStruct((NPAD,), _F32)],
    mesh=_MESH,
    compiler_params=_SC_PARAMS,
    scratch_types=[
        pltpu.VMEM((CPT, CH), jnp.int32),
        pltpu.VMEM((CPT, CH), _F32),
        pltpu.VMEM((1024,), _F32),
        pltpu.VMEM_SHARED((NPAD,), _F32),
    ],
)


def _scale_scatter(rows_bf, rowsf, ewst, acc, didx_v, g):
    for j in range(CH // 16):
        nv = ewst[g, pl.ds(j * 16, 16)]
        for t in range(16):
            e = j * 16 + t
            ne = nv[t]
            for q in range(4):
                ab = plsc.bitcast(rows_bf[e, pl.ds(q * 16, 16)], _BF16)
                a, b = plsc.unpack(ab, format=plsc.PackFormat.INTERLEAVED)
                rowsf[e, pl.ds(q * 32, 16)] = a * ne
                rowsf[e, pl.ds(q * 32 + 16, 16)] = b * ne
    pltpu.sync_copy(rowsf, acc.at[didx_v.at[g]], add=True)


def _scat_body(hsb, srcr, dstr, ewr, part0, part1,
               sidx_v, didx_v, ewst, rows_a, rows_b, rowsf, acc,
               sem_a, sem_b):
    cid = lax.axis_index("c")
    sid = lax.axis_index("s")
    wid = cid * 16 + sid
    z16 = jnp.zeros((16,), _F32)

    def zrow(i, carry):
        for q in range(8):
            rowsf[i, pl.ds(q * 16, 16)] = z16
        return carry

    lax.fori_loop(0, CH, zrow, 0)
    for k in range(10):
        pltpu.sync_copy(rowsf, acc.at[pl.ds(sid * 640 + k * CH, CH)])
    plsc.subcore_barrier()

    HC = CPT // 2
    for half in range(2):
        pltpu.sync_copy(srcr.at[wid, pl.ds(half * HC, HC)], sidx_v)
        pltpu.sync_copy(dstr.at[wid, pl.ds(half * HC, HC)], didx_v)
        pltpu.sync_copy(ewr.at[wid, pl.ds(half * HC, HC)], ewst)
        pltpu.async_copy(hsb.at[sidx_v.at[0]], rows_a, sem_a)

        def pair(p, carry):
            g0 = 2 * p
            g1 = g0 + 1
            pltpu.make_async_copy(hsb.at[sidx_v.at[g0]], rows_a,
                                  sem_a).wait()
            pltpu.async_copy(hsb.at[sidx_v.at[g1]], rows_b, sem_b)
            _scale_scatter(rows_a, rowsf, ewst, acc, didx_v, g0)
            pltpu.make_async_copy(hsb.at[sidx_v.at[g1]], rows_b,
                                  sem_b).wait()

            @pl.when(g0 + 2 < HC)
            def _():
                pltpu.async_copy(hsb.at[sidx_v.at[g0 + 2]], rows_a, sem_a)

            _scale_scatter(rows_b, rowsf, ewst, acc, didx_v, g1)
            return carry

        lax.fori_loop(0, HC // 2, pair, 0)
    plsc.subcore_barrier()

    @pl.when(cid == 0)
    def _():
        pltpu.sync_copy(acc.at[pl.ds(sid * 640, 640)],
                        part0.at[pl.ds(sid * 640, 640)])

    @pl.when(cid == 1)
    def _():
        pltpu.sync_copy(acc.at[pl.ds(sid * 640, 640)],
                        part1.at[pl.ds(sid * 640, 640)])


_scat_call = pl.kernel(
    _scat_body,
    out_type=[jax.ShapeDtypeStruct((NROWS, D), _F32),
              jax.ShapeDtypeStruct((NROWS, D), _F32)],
    mesh=_MESH,
    compiler_params=_SC_PARAMS,
    scratch_types=[
        pltpu.VMEM((CPT // 2, CH), jnp.int32),
        pltpu.VMEM((CPT // 2, CH), jnp.int32),
        pltpu.VMEM((CPT // 2, CH), _F32),
        pltpu.VMEM((CH, D // 2), jnp.int32),
        pltpu.VMEM((CH, D // 2), jnp.int32),
        pltpu.VMEM((CH, D), _F32),
        pltpu.VMEM_SHARED((NROWS, D), _F32),
        pltpu.SemaphoreType.DMA,
        pltpu.SemaphoreType.DMA,
    ],
)


def _h0_body(x_ref, w_ref, wp_ref, deg0_ref, deg1_ref,
             hs_ref, hsb_ref, dinv_ref):
    d = deg0_ref[...] + deg1_ref[...]
    dv = jnp.where(d > 0, lax.rsqrt(d), 0.0)
    dinv_ref[...] = dv
    hs_ref[...] = jnp.dot(x_ref[...], w_ref[...],
                          preferred_element_type=_F32) * dv
    hsb_ref[...] = (jnp.dot(x_ref[...], wp_ref[...],
                            preferred_element_type=_F32) * dv).astype(_BF16)


_h0_call = pl.pallas_call(
    _h0_body,
    out_shape=[jax.ShapeDtypeStruct((N, D), _F32),
               jax.ShapeDtypeStruct((N, D), _BF16),
               jax.ShapeDtypeStruct((N, 1), _F32)],
)


def _epi1_body(p0_ref, p1_ref, hs_ref, dinv_ref, b_ref, g_ref, bt_ref,
               w_ref, wp_ref, hs1_ref, hs1b_ref):
    dv = dinv_ref[...]
    y = (p0_ref[0:N, :] + p1_ref[0:N, :] + hs_ref[...]) * dv + b_ref[...]
    y = jnp.maximum(y, 0.0)
    m = jnp.mean(y, axis=0, keepdims=True)
    v = jnp.mean(y * y, axis=0, keepdims=True) - m * m
    a = g_ref[...] * lax.rsqrt(v + EPS)
    z = (y - m) * a + bt_ref[...]
    hs1_ref[...] = jnp.dot(z, w_ref[...], preferred_element_type=_F32) * dv
    hs1b_ref[...] = (jnp.dot(z, wp_ref[...],
                             preferred_element_type=_F32) * dv).astype(_BF16)


_epi1_call = pl.pallas_call(
    _epi1_body,
    out_shape=[jax.ShapeDtypeStruct((N, D), _F32),
               jax.ShapeDtypeStruct((N, D), _BF16)],
)


def _epi2_body(p0_ref, p1_ref, hs_ref, dinv_ref, b_ref, g_ref, bt_ref,
               batch_ref, wl1_ref, bl1_ref, wl2_ref, bl2_ref, out_ref):
    dv = dinv_ref[...]
    y = (p0_ref[0:N, :] + p1_ref[0:N, :] + hs_ref[...]) * dv + b_ref[...]
    y = jnp.maximum(y, 0.0)
    m = jnp.mean(y, axis=0, keepdims=True)
    v = jnp.mean(y * y, axis=0, keepdims=True) - m * m
    a = g_ref[...] * lax.rsqrt(v + EPS)
    z = (y - m) * a + bt_ref[...]
    oh = (batch_ref[...] == lax.broadcasted_iota(jnp.int32, (G, 1), 0))
    pooled = jnp.dot(oh.astype(_F32), z, preferred_element_type=_F32,
                     precision=lax.Precision.HIGHEST)
    h2 = jnp.maximum(
        jnp.dot(pooled, wl1_ref[...], preferred_element_type=_F32)
        + bl1_ref[...], 0.0)
    logits = (jnp.dot(h2, wl2_ref[...], preferred_element_type=_F32)
              + bl2_ref[...])
    mx = jnp.max(logits, axis=-1, keepdims=True)
    sh = logits - mx
    out_ref[...] = sh - jnp.log(jnp.sum(jnp.exp(sh), axis=-1, keepdims=True))


_epi2_call = pl.pallas_call(
    _epi2_body,
    out_shape=jax.ShapeDtypeStruct((G, C), _F32),
)


def kernel(x, edge_index, batch, edge_attr,
           W0, b0, W1, b1, g0, bt0, g1, bt1, Wl1, bl1, Wl2, bl2):
    src = edge_index[0].astype(jnp.int32)
    dst = edge_index[1].astype(jnp.int32)
    ew = edge_attr[:, 0]
    pad = EPADT - E
    pad_idx = (jnp.arange(pad, dtype=jnp.int32) * 37) % N
    src_p = jnp.concatenate([src, pad_idx])
    dst_p = jnp.concatenate([dst, pad_idx])
    ew_p = jnp.concatenate([ew, jnp.zeros((pad,), _F32)])
    srcr = src_p.reshape(32, CPT, CH)
    dstr = dst_p.reshape(32, CPT, CH)
    ewr = ew_p.reshape(32, CPT, CH)

    W0p = W0[:, PHI]
    W1p = W1[:, PHI]

    deg0, deg1 = _deg_call(dstr, ewr)
    hs0, hs0b, dinv_col = _h0_call(x, W0, W0p, deg0[:N].reshape(N, 1),
                                   deg1[:N].reshape(N, 1))

    def _as_i32(hb):
        return lax.bitcast_convert_type(hb.reshape(N, D // 2, 2), jnp.int32)

    p0, p1 = _scat_call(_as_i32(hs0b), srcr, dstr, ewr)
    hs1, hs1b = _epi1_call(p0, p1, hs0, dinv_col, b0.reshape(1, D),
                           g0.reshape(1, D), bt0.reshape(1, D), W1, W1p)
    q0, q1 = _scat_call(_as_i32(hs1b), srcr, dstr, ewr)
    out = _epi2_call(q0, q1, hs1, dinv_col, b1.reshape(1, D),
                     g1.reshape(1, D), bt1.reshape(1, D),
                     batch.astype(jnp.int32).reshape(1, N),
                     Wl1, bl1.reshape(1, D), Wl2, bl2.reshape(1, C))
    return out
